# baseline (speedup 1.0000x reference)
import functools

import jax
import jax.numpy as jnp
from jax import lax
from jax.experimental import pallas as pl
from jax.experimental.pallas import tpu as pltpu
from jax.experimental.pallas import tpu_sc as plsc

NC = 2
NS = 16
CHUNK = 128


def _round_up(v, m):
    return (v + m - 1) // m * m


def _make_deg_kernel(n_pad, tot, zp, zpb):
    rt = tot // NS
    mesh = plsc.VectorSubcoreMesh(core_axis_name="c", subcore_axis_name="s")

    @functools.partial(
        pl.kernel,
        out_type=jax.ShapeDtypeStruct((NC * n_pad,), jnp.float32),
        mesh=mesh,
        scratch_types=[
            pltpu.VMEM((rt, CHUNK), jnp.int32),
            pltpu.VMEM((CHUNK,), jnp.float32),
            pltpu.VMEM((zpb,), jnp.float32),
            pltpu.VMEM_SHARED((n_pad,), jnp.float32),
        ],
    )
    def deg_kernel(ecat_hbm, z1_hbm, out_hbm, idx_v, ones_v, zv, acc_s):
        c = lax.axis_index("c")
        s = lax.axis_index("s")
        for j in range(CHUNK // 16):
            ones_v[pl.ds(j * 16, 16)] = jnp.ones((16,), jnp.float32)
        pltpu.sync_copy(z1_hbm, zv)
        base = s * zp
        pltpu.sync_copy(zv.at[pl.ds(0, zp)], acc_s.at[pl.ds(base, zp)])
        pltpu.sync_copy(ecat_hbm.at[pl.ds(c * tot + s * rt, rt)], idx_v)
        plsc.subcore_barrier()

        def body(i, carry):
            pltpu.sync_copy(ones_v, acc_s.at[idx_v.at[i]], add=True)
            return carry

        lax.fori_loop(0, rt, body, 0)
        plsc.subcore_barrier()
        pltpu.sync_copy(acc_s.at[pl.ds(base, zp)], zv.at[pl.ds(0, zp)])
        pltpu.sync_copy(zv.at[pl.ds(0, zp)],
                        out_hbm.at[pl.ds(c * n_pad + base, zp)])

    return deg_kernel


def _make_edge_kernel(n_pad, tot, rw, zp, zpb, d, ec, with_u):
    mesh = plsc.VectorSubcoreMesh(core_axis_name="c", subcore_axis_name="s")
    scratch = [
        pltpu.VMEM((rw, 2 * ec), jnp.int32),
        pltpu.VMEM((rw, 2 * ec), jnp.int32),
        pltpu.VMEM((ec, d), jnp.float32),
        pltpu.VMEM((ec, d), jnp.float32),
        pltpu.VMEM((8, d), jnp.float32),
        pltpu.SemaphoreType.DMA,
        pltpu.SemaphoreType.DMA,
        pltpu.SemaphoreType.DMA,
        pltpu.SemaphoreType.DMA,
        pltpu.VMEM_SHARED((n_pad, d), jnp.float32),
    ]
    out_type = [jax.ShapeDtypeStruct((NC * n_pad, d), jnp.float32)]
    if with_u:
        scratch += [
            pltpu.VMEM((ec,), jnp.float32),
            pltpu.VMEM((ec,), jnp.float32),
            pltpu.VMEM((zpb,), jnp.float32),
            pltpu.SemaphoreType.DMA,
            pltpu.SemaphoreType.DMA,
            pltpu.SemaphoreType.DMA,
            pltpu.SemaphoreType.DMA,
            pltpu.VMEM_SHARED((n_pad,), jnp.float32),
        ]
        out_type.append(jax.ShapeDtypeStruct((NC * n_pad,), jnp.float32))

    mk = functools.partial(pl.kernel, out_type=tuple(out_type), mesh=mesh,
                           scratch_types=scratch)

    def common(c, s, g_hbm, src_hbm, dst_hbm, z2_hbm, agg_hbm, src_v, dst_v,
               bufs, zrow, gsems, ssems, acc_s, uprefetch, uconsume, urefill,
               ufin):
        w = s * NC + c
        base = s * zp
        pltpu.sync_copy(z2_hbm, zrow)

        def zbody(k, carry):
            pltpu.sync_copy(zrow, acc_s.at[pl.ds(base + k * 8, 8)])
            return carry

        lax.fori_loop(0, zp // 8, zbody, 0)
        pltpu.sync_copy(src_hbm.at[pl.ds(w * rw, rw)], src_v)
        pltpu.sync_copy(dst_hbm.at[pl.ds(w * rw, rw)], dst_v)
        plsc.subcore_barrier()

        def sl(ref, i, p):
            return ref.at[i, pl.ds(p * ec, ec)]

        def gath(i, p):
            pltpu.async_copy(g_hbm.at[sl(src_v, i, p)], bufs[p], gsems[p])

        def wait_g(p):
            pltpu.make_async_copy(g_hbm.at[sl(src_v, 0, p)], bufs[p],
                                  gsems[p]).wait()

        def scat(i, p):
            pltpu.async_copy(bufs[p], acc_s.at[sl(dst_v, i, p)], ssems[p],
                             add=True)

        def wait_s(p):
            pltpu.make_async_copy(bufs[p], acc_s.at[sl(dst_v, 0, p)],
                                  ssems[p]).wait()

        gath(0, 0)
        gath(0, 1)
        uprefetch(0, 0)
        uprefetch(0, 1)

        def body(k, carry):
            nxt = jnp.minimum(k + 1, rw - 1)
            wait_g(0)
            scat(k, 0)
            wait_g(1)
            scat(k, 1)
            uconsume(k, 0)
            uconsume(k, 1)
            wait_s(0)
            gath(nxt, 0)
            wait_s(1)
            gath(nxt, 1)
            urefill(nxt, 0)
            urefill(nxt, 1)
            return carry

        lax.fori_loop(0, rw, body, 0)
        for p in (0, 1):
            wait_g(p)
        plsc.subcore_barrier()
        nfull = zp // ec
        rem = zp - nfull * ec
        for k in range(nfull + 1):
            span = ec if k < nfull else rem
            if span:
                pltpu.sync_copy(acc_s.at[pl.ds(base + k * ec, span)],
                                bufs[0].at[pl.ds(0, span)])
                pltpu.sync_copy(bufs[0].at[pl.ds(0, span)],
                                agg_hbm.at[pl.ds(c * n_pad + base + k * ec,
                                                 span)])
        ufin(c, base)

    if with_u:
        @mk
        def edge_kernel(g_hbm, src_hbm, dst_hbm, z2_hbm, b_hbm, z1_hbm,
                        agg_hbm, u_hbm, src_v, dst_v, bufa, bufb, zrow,
                        ga, gb, sa, sb, acc_s, ubufa, ubufb, zv,
                        uga, ugb, usa, usb, uacc_s):
            c = lax.axis_index("c")
            s = lax.axis_index("s")
            base = s * zp
            ubufs, ugs, uss = (ubufa, ubufb), (uga, ugb), (usa, usb)
            pltpu.sync_copy(z1_hbm, zv)
            pltpu.sync_copy(zv.at[pl.ds(0, zp)], uacc_s.at[pl.ds(base, zp)])

            def usl(ref, i, p):
                return ref.at[i, pl.ds(p * ec, ec)]

            def uprefetch(i, p):
                pltpu.async_copy(b_hbm.at[usl(dst_v, i, p)], ubufs[p], ugs[p])

            def uconsume(i, p):
                pltpu.make_async_copy(b_hbm.at[usl(dst_v, i, p)], ubufs[p],
                                      ugs[p]).wait()
                pltpu.async_copy(ubufs[p], uacc_s.at[usl(src_v, i, p)],
                                 uss[p], add=True)

            def urefill(i, p):
                pltpu.make_async_copy(ubufs[p], uacc_s.at[usl(src_v, i, p)],
                                      uss[p]).wait()
                uprefetch(i, p)

            def ufin(cc, bb):
                pltpu.sync_copy(uacc_s.at[pl.ds(bb, zp)], zv.at[pl.ds(0, zp)])
                pltpu.sync_copy(zv.at[pl.ds(0, zp)],
                                u_hbm.at[pl.ds(cc * n_pad + bb, zp)])

            common(c, s, g_hbm, src_hbm, dst_hbm, z2_hbm, agg_hbm,
                   src_v, dst_v, (bufa, bufb), zrow, (ga, gb), (sa, sb),
                   acc_s, uprefetch, uconsume, urefill, ufin)
            for p in (0, 1):
                pltpu.make_async_copy(b_hbm.at[usl(dst_v, 0, p)], ubufs[p],
                                      ugs[p]).wait()
    else:
        @mk
        def edge_kernel(g_hbm, src_hbm, dst_hbm, z2_hbm, agg_hbm,
                        src_v, dst_v, bufa, bufb, zrow, ga, gb, sa, sb,
                        acc_s):
            c = lax.axis_index("c")
            s = lax.axis_index("s")
            common(c, s, g_hbm, src_hbm, dst_hbm, z2_hbm, agg_hbm,
                   src_v, dst_v, (bufa, bufb), zrow, (ga, gb), (sa, sb),
                   acc_s, lambda i, p: None, lambda i, p: None,
                   lambda i, p: None, lambda cc, bb: None)

    return edge_kernel


def _prep_body(n_real, dego_ref, degi_ref, x_ref, win_ref, bin_ref, w1_ref,
               g_ref, a_ref, b_ref):
    i = pl.program_id(0)
    blk = dego_ref.shape[0]
    a = lax.rsqrt(jnp.maximum(dego_ref[...], 1.0))
    rows = lax.broadcasted_iota(jnp.int32, (blk, 1), 0) + i * blk
    b = jnp.where(rows < n_real,
                  lax.rsqrt(jnp.maximum(degi_ref[...], 1.0)), 0.0)
    t = jnp.dot(x_ref[...], win_ref[...],
                preferred_element_type=jnp.float32) + bin_ref[...]
    g = jnp.dot(t, w1_ref[...], preferred_element_type=jnp.float32)
    g_ref[...] = a * g
    a_ref[...] = a
    b_ref[...] = b


def _mid_body(agg_ref, a_ref, b_ref, b1_ref, w2_ref, g_ref):
    ssum = agg_ref[0] + agg_ref[1]
    h = jnp.maximum(b_ref[...] * ssum + b1_ref[...], 0.0)
    g_ref[...] = a_ref[...] * jnp.dot(h, w2_ref[...],
                                      preferred_element_type=jnp.float32)


def _final_body(n_real, agg_ref, u_ref, a_ref, b_ref, b2_ref, w3_ref, b3_ref,
                wo_ref, bo_ref, out_ref, acc_ref):
    i = pl.program_id(0)

    @pl.when(i == 0)
    def _():
        acc_ref[...] = jnp.zeros_like(acc_ref)

    ssum = agg_ref[0] + agg_ref[1]
    h2 = jnp.maximum(b_ref[...] * ssum + b2_ref[...], 0.0)
    wv = a_ref[...] * (u_ref[0] + u_ref[1])
    acc_ref[0:1, :] += jnp.sum(wv * h2, axis=0, keepdims=True)

    @pl.when(i == pl.num_programs(0) - 1)
    def _():
        p = acc_ref[0:1, :]
        t = (jnp.dot(p, w3_ref[...], preferred_element_type=jnp.float32)
             + jnp.float32(n_real) * b3_ref[...])
        out_ref[...] = (jnp.dot(t, wo_ref[...],
                                preferred_element_type=jnp.float32)
                        + bo_ref[...])


def kernel(x, edge_index, W_in, b_in, W1, b1, W2, b2, W3, b3, W_out, b_out):
    n, d = x.shape
    e = edge_index.shape[1]
    nw = NC * NS
    e_pad = _round_up(e, nw * CHUNK * 8)
    tot = e_pad // CHUNK
    ec = 64
    rw = tot // nw
    n_pad = _round_up(n + 16, 128)
    zp = n_pad // NS
    zpb = _round_up(zp, 16)
    blk = 128
    grid = n_pad // blk

    f32 = jnp.float32
    src = edge_index[0]
    dst = edge_index[1]
    padc = e_pad - e
    if padc:
        pidx = (n + (jnp.arange(padc, dtype=jnp.int32) % 16)).astype(jnp.int32)
        src = jnp.concatenate([src, pidx])
        dst = jnp.concatenate([dst, pidx])
    src2 = src.reshape(tot, CHUNK)
    dst2 = dst.reshape(tot, CHUNK)
    ecat = jnp.concatenate([src2, dst2], axis=0)
    z2 = jnp.zeros((8, d), f32)
    z1 = jnp.zeros((zpb,), f32)
    x_pad = jnp.concatenate([x, jnp.zeros((n_pad - n, d), f32)], axis=0)

    deg = _make_deg_kernel(n_pad, tot, zp, zpb)(ecat, z1)
    dego = deg[:n_pad].reshape(n_pad, 1)
    degi = deg[n_pad:].reshape(n_pad, 1)

    col_spec = pl.BlockSpec((blk, 1), lambda i: (i, 0))
    row_spec = pl.BlockSpec((blk, d), lambda i: (i, 0))
    mat_spec = pl.BlockSpec((d, d), lambda i: (0, 0))
    vec_spec = pl.BlockSpec((1, d), lambda i: (0, 0))
    g0p, a_col, b_col = pl.pallas_call(
        functools.partial(_prep_body, n),
        grid=(grid,),
        in_specs=[col_spec, col_spec, row_spec, mat_spec, vec_spec, mat_spec],
        out_specs=[row_spec, col_spec, col_spec],
        out_shape=[jax.ShapeDtypeStruct((n_pad, d), f32),
                   jax.ShapeDtypeStruct((n_pad, 1), f32),
                   jax.ShapeDtypeStruct((n_pad, 1), f32)],
    )(dego, degi, x_pad, W_in, b_in.reshape(1, d), W1)

    agg1, u = _make_edge_kernel(n_pad, tot, rw, zp, zpb, d, ec, True)(
        g0p, src2, dst2, z2, b_col.reshape(n_pad), z1)

    agg_spec = pl.BlockSpec((2, blk, d), lambda i: (0, i, 0))
    g1p = pl.pallas_call(
        _mid_body,
        grid=(grid,),
        in_specs=[agg_spec, col_spec, col_spec, vec_spec, mat_spec],
        out_specs=row_spec,
        out_shape=jax.ShapeDtypeStruct((n_pad, d), f32),
    )(agg1.reshape(NC, n_pad, d), a_col, b_col, b1.reshape(1, d), W2)

    (agg2,) = _make_edge_kernel(n_pad, tot, rw, zp, zpb, d, ec, False)(
        g1p, src2, dst2, z2)

    u_spec = pl.BlockSpec((2, blk, 1), lambda i: (0, i, 0))
    out = pl.pallas_call(
        functools.partial(_final_body, n),
        grid=(grid,),
        in_specs=[agg_spec, u_spec, col_spec, col_spec, vec_spec, mat_spec,
                  vec_spec, mat_spec, vec_spec],
        out_specs=pl.BlockSpec((1, d), lambda i: (0, 0)),
        out_shape=jax.ShapeDtypeStruct((1, d), f32),
        scratch_shapes=[pltpu.VMEM((8, d), f32)],
    )(agg2.reshape(NC, n_pad, d), u.reshape(NC, n_pad, 1), a_col, b_col,
      b2.reshape(1, d), W3, b3.reshape(1, d), W_out, b_out.reshape(1, d))
    return out.reshape(d)


# speedup vs baseline: 1.0056x; 1.0056x over previous
import functools

import jax
import jax.numpy as jnp
from jax import lax
from jax.experimental import pallas as pl
from jax.experimental.pallas import tpu as pltpu
from jax.experimental.pallas import tpu_sc as plsc

NC = 2
NS = 16
CHUNK = 128


def _round_up(v, m):
    return (v + m - 1) // m * m


def _make_deg_kernel(n_pad, tot, zp, zpb):
    rt = tot // NS
    mesh = plsc.VectorSubcoreMesh(core_axis_name="c", subcore_axis_name="s")

    @functools.partial(
        pl.kernel,
        out_type=jax.ShapeDtypeStruct((NC * n_pad,), jnp.float32),
        mesh=mesh,
        scratch_types=[
            pltpu.VMEM((rt, CHUNK), jnp.int32),
            pltpu.VMEM((CHUNK,), jnp.float32),
            pltpu.VMEM((zpb,), jnp.float32),
            pltpu.VMEM_SHARED((n_pad,), jnp.float32),
        ],
    )
    def deg_kernel(ecat_hbm, z1_hbm, out_hbm, idx_v, ones_v, zv, acc_s):
        c = lax.axis_index("c")
        s = lax.axis_index("s")
        for j in range(CHUNK // 16):
            ones_v[pl.ds(j * 16, 16)] = jnp.ones((16,), jnp.float32)
        pltpu.sync_copy(z1_hbm, zv)
        base = s * zp
        pltpu.sync_copy(zv.at[pl.ds(0, zp)], acc_s.at[pl.ds(base, zp)])
        pltpu.sync_copy(ecat_hbm.at[pl.ds(c * tot + s * rt, rt)], idx_v)
        plsc.subcore_barrier()

        def body(i, carry):
            pltpu.sync_copy(ones_v, acc_s.at[idx_v.at[i]], add=True)
            return carry

        lax.fori_loop(0, rt, body, 0)
        plsc.subcore_barrier()
        pltpu.sync_copy(acc_s.at[pl.ds(base, zp)], zv.at[pl.ds(0, zp)])
        pltpu.sync_copy(zv.at[pl.ds(0, zp)],
                        out_hbm.at[pl.ds(c * n_pad + base, zp)])

    return deg_kernel


def _make_edge_kernel(n_pad, tot, rw, zp, zpb, d, ec, with_u):
    mesh = plsc.VectorSubcoreMesh(core_axis_name="c", subcore_axis_name="s")
    scratch = [
        pltpu.VMEM((rw, 2 * ec), jnp.int32),
        pltpu.VMEM((rw, 2 * ec), jnp.int32),
        pltpu.VMEM((ec, d), jnp.float32),
        pltpu.VMEM((ec, d), jnp.float32),
        pltpu.VMEM((8, d), jnp.float32),
        pltpu.SemaphoreType.DMA,
        pltpu.SemaphoreType.DMA,
        pltpu.SemaphoreType.DMA,
        pltpu.SemaphoreType.DMA,
        pltpu.VMEM_SHARED((n_pad, d), jnp.float32),
    ]
    out_type = [jax.ShapeDtypeStruct((NC * n_pad, d), jnp.float32)]
    if with_u:
        scratch += [
            pltpu.VMEM((ec,), jnp.float32),
            pltpu.VMEM((ec,), jnp.float32),
            pltpu.VMEM((zpb,), jnp.float32),
            pltpu.SemaphoreType.DMA,
            pltpu.SemaphoreType.DMA,
            pltpu.SemaphoreType.DMA,
            pltpu.SemaphoreType.DMA,
            pltpu.VMEM_SHARED((n_pad,), jnp.float32),
        ]
        out_type.append(jax.ShapeDtypeStruct((NC * n_pad,), jnp.float32))

    mk = functools.partial(pl.kernel, out_type=tuple(out_type), mesh=mesh,
                           scratch_types=scratch)

    def common(c, s, g_hbm, src_hbm, dst_hbm, z2_hbm, agg_hbm, src_v, dst_v,
               bufs, zrow, gsems, ssems, acc_s, uprefetch, uconsume, urefill,
               ufin):
        w = s * NC + c
        base = s * zp
        pltpu.sync_copy(z2_hbm, zrow)

        def zbody(k, carry):
            pltpu.sync_copy(zrow, acc_s.at[pl.ds(base + k * 8, 8)])
            return carry

        lax.fori_loop(0, zp // 8, zbody, 0)
        pltpu.sync_copy(src_hbm.at[pl.ds(w * rw, rw)], src_v)
        pltpu.sync_copy(dst_hbm.at[pl.ds(w * rw, rw)], dst_v)
        plsc.subcore_barrier()

        def sl(ref, i, p):
            return ref.at[i, pl.ds(p * ec, ec)]

        def gath(i, p):
            pltpu.async_copy(g_hbm.at[sl(src_v, i, p)], bufs[p], gsems[p])

        def wait_g(p):
            pltpu.make_async_copy(g_hbm.at[sl(src_v, 0, p)], bufs[p],
                                  gsems[p]).wait()

        def scat(i, p):
            pltpu.async_copy(bufs[p], acc_s.at[sl(dst_v, i, p)], ssems[p],
                             add=True)

        def wait_s(p):
            pltpu.make_async_copy(bufs[p], acc_s.at[sl(dst_v, 0, p)],
                                  ssems[p]).wait()

        gath(0, 0)
        gath(0, 1)
        uprefetch(0, 0)
        uprefetch(0, 1)

        def body(k, carry):
            nxt = jnp.minimum(k + 1, rw - 1)
            wait_g(0)
            scat(k, 0)
            wait_g(1)
            scat(k, 1)
            uconsume(k, 0)
            wait_s(0)
            gath(nxt, 0)
            uconsume(k, 1)
            wait_s(1)
            gath(nxt, 1)
            urefill(nxt, 0)
            urefill(nxt, 1)
            return carry

        lax.fori_loop(0, rw, body, 0)
        for p in (0, 1):
            wait_g(p)
        plsc.subcore_barrier()
        nfull = zp // ec
        rem = zp - nfull * ec
        for k in range(nfull + 1):
            span = ec if k < nfull else rem
            if span:
                pltpu.sync_copy(acc_s.at[pl.ds(base + k * ec, span)],
                                bufs[0].at[pl.ds(0, span)])
                pltpu.sync_copy(bufs[0].at[pl.ds(0, span)],
                                agg_hbm.at[pl.ds(c * n_pad + base + k * ec,
                                                 span)])
        ufin(c, base)

    if with_u:
        @mk
        def edge_kernel(g_hbm, src_hbm, dst_hbm, z2_hbm, b_hbm, z1_hbm,
                        agg_hbm, u_hbm, src_v, dst_v, bufa, bufb, zrow,
                        ga, gb, sa, sb, acc_s, ubufa, ubufb, zv,
                        uga, ugb, usa, usb, uacc_s):
            c = lax.axis_index("c")
            s = lax.axis_index("s")
            base = s * zp
            ubufs, ugs, uss = (ubufa, ubufb), (uga, ugb), (usa, usb)
            pltpu.sync_copy(z1_hbm, zv)
            pltpu.sync_copy(zv.at[pl.ds(0, zp)], uacc_s.at[pl.ds(base, zp)])

            def usl(ref, i, p):
                return ref.at[i, pl.ds(p * ec, ec)]

            def uprefetch(i, p):
                pltpu.async_copy(b_hbm.at[usl(dst_v, i, p)], ubufs[p], ugs[p])

            def uconsume(i, p):
                pltpu.make_async_copy(b_hbm.at[usl(dst_v, i, p)], ubufs[p],
                                      ugs[p]).wait()
                pltpu.async_copy(ubufs[p], uacc_s.at[usl(src_v, i, p)],
                                 uss[p], add=True)

            def urefill(i, p):
                pltpu.make_async_copy(ubufs[p], uacc_s.at[usl(src_v, i, p)],
                                      uss[p]).wait()
                uprefetch(i, p)

            def ufin(cc, bb):
                pltpu.sync_copy(uacc_s.at[pl.ds(bb, zp)], zv.at[pl.ds(0, zp)])
                pltpu.sync_copy(zv.at[pl.ds(0, zp)],
                                u_hbm.at[pl.ds(cc * n_pad + bb, zp)])

            common(c, s, g_hbm, src_hbm, dst_hbm, z2_hbm, agg_hbm,
                   src_v, dst_v, (bufa, bufb), zrow, (ga, gb), (sa, sb),
                   acc_s, uprefetch, uconsume, urefill, ufin)
            for p in (0, 1):
                pltpu.make_async_copy(b_hbm.at[usl(dst_v, 0, p)], ubufs[p],
                                      ugs[p]).wait()
    else:
        @mk
        def edge_kernel(g_hbm, src_hbm, dst_hbm, z2_hbm, agg_hbm,
                        src_v, dst_v, bufa, bufb, zrow, ga, gb, sa, sb,
                        acc_s):
            c = lax.axis_index("c")
            s = lax.axis_index("s")
            common(c, s, g_hbm, src_hbm, dst_hbm, z2_hbm, agg_hbm,
                   src_v, dst_v, (bufa, bufb), zrow, (ga, gb), (sa, sb),
                   acc_s, lambda i, p: None, lambda i, p: None,
                   lambda i, p: None, lambda cc, bb: None)

    return edge_kernel


def _prep_body(n_real, dego_ref, degi_ref, x_ref, win_ref, bin_ref, w1_ref,
               g_ref, a_ref, b_ref):
    i = pl.program_id(0)
    blk = dego_ref.shape[0]
    a = lax.rsqrt(jnp.maximum(dego_ref[...], 1.0))
    rows = lax.broadcasted_iota(jnp.int32, (blk, 1), 0) + i * blk
    b = jnp.where(rows < n_real,
                  lax.rsqrt(jnp.maximum(degi_ref[...], 1.0)), 0.0)
    t = jnp.dot(x_ref[...], win_ref[...],
                preferred_element_type=jnp.float32) + bin_ref[...]
    g = jnp.dot(t, w1_ref[...], preferred_element_type=jnp.float32)
    g_ref[...] = a * g
    a_ref[...] = a
    b_ref[...] = b


def _mid_body(agg_ref, a_ref, b_ref, b1_ref, w2_ref, g_ref):
    ssum = agg_ref[0] + agg_ref[1]
    h = jnp.maximum(b_ref[...] * ssum + b1_ref[...], 0.0)
    g_ref[...] = a_ref[...] * jnp.dot(h, w2_ref[...],
                                      preferred_element_type=jnp.float32)


def _final_body(n_real, agg_ref, u_ref, a_ref, b_ref, b2_ref, w3_ref, b3_ref,
                wo_ref, bo_ref, out_ref, acc_ref):
    i = pl.program_id(0)

    @pl.when(i == 0)
    def _():
        acc_ref[...] = jnp.zeros_like(acc_ref)

    ssum = agg_ref[0] + agg_ref[1]
    h2 = jnp.maximum(b_ref[...] * ssum + b2_ref[...], 0.0)
    wv = a_ref[...] * (u_ref[0] + u_ref[1])
    acc_ref[0:1, :] += jnp.sum(wv * h2, axis=0, keepdims=True)

    @pl.when(i == pl.num_programs(0) - 1)
    def _():
        p = acc_ref[0:1, :]
        t = (jnp.dot(p, w3_ref[...], preferred_element_type=jnp.float32)
             + jnp.float32(n_real) * b3_ref[...])
        out_ref[...] = (jnp.dot(t, wo_ref[...],
                                preferred_element_type=jnp.float32)
                        + bo_ref[...])


def kernel(x, edge_index, W_in, b_in, W1, b1, W2, b2, W3, b3, W_out, b_out):
    n, d = x.shape
    e = edge_index.shape[1]
    nw = NC * NS
    e_pad = _round_up(e, nw * CHUNK * 8)
    tot = e_pad // CHUNK
    ec = 64
    rw = tot // nw
    n_pad = _round_up(n + 16, 128)
    zp = n_pad // NS
    zpb = _round_up(zp, 16)
    blk = 128
    grid = n_pad // blk

    f32 = jnp.float32
    src = edge_index[0]
    dst = edge_index[1]
    padc = e_pad - e
    if padc:
        pidx = (n + (jnp.arange(padc, dtype=jnp.int32) % 16)).astype(jnp.int32)
        src = jnp.concatenate([src, pidx])
        dst = jnp.concatenate([dst, pidx])
    src2 = src.reshape(tot, CHUNK)
    dst2 = dst.reshape(tot, CHUNK)
    ecat = jnp.concatenate([src2, dst2], axis=0)
    z2 = jnp.zeros((8, d), f32)
    z1 = jnp.zeros((zpb,), f32)
    x_pad = jnp.concatenate([x, jnp.zeros((n_pad - n, d), f32)], axis=0)

    deg = _make_deg_kernel(n_pad, tot, zp, zpb)(ecat, z1)
    dego = deg[:n_pad].reshape(n_pad, 1)
    degi = deg[n_pad:].reshape(n_pad, 1)

    col_spec = pl.BlockSpec((blk, 1), lambda i: (i, 0))
    row_spec = pl.BlockSpec((blk, d), lambda i: (i, 0))
    mat_spec = pl.BlockSpec((d, d), lambda i: (0, 0))
    vec_spec = pl.BlockSpec((1, d), lambda i: (0, 0))
    g0p, a_col, b_col = pl.pallas_call(
        functools.partial(_prep_body, n),
        grid=(grid,),
        in_specs=[col_spec, col_spec, row_spec, mat_spec, vec_spec, mat_spec],
        out_specs=[row_spec, col_spec, col_spec],
        out_shape=[jax.ShapeDtypeStruct((n_pad, d), f32),
                   jax.ShapeDtypeStruct((n_pad, 1), f32),
                   jax.ShapeDtypeStruct((n_pad, 1), f32)],
    )(dego, degi, x_pad, W_in, b_in.reshape(1, d), W1)

    agg1, u = _make_edge_kernel(n_pad, tot, rw, zp, zpb, d, ec, True)(
        g0p, src2, dst2, z2, b_col.reshape(n_pad), z1)

    agg_spec = pl.BlockSpec((2, blk, d), lambda i: (0, i, 0))
    g1p = pl.pallas_call(
        _mid_body,
        grid=(grid,),
        in_specs=[agg_spec, col_spec, col_spec, vec_spec, mat_spec],
        out_specs=row_spec,
        out_shape=jax.ShapeDtypeStruct((n_pad, d), f32),
    )(agg1.reshape(NC, n_pad, d), a_col, b_col, b1.reshape(1, d), W2)

    (agg2,) = _make_edge_kernel(n_pad, tot, rw, zp, zpb, d, ec, False)(
        g1p, src2, dst2, z2)

    u_spec = pl.BlockSpec((2, blk, 1), lambda i: (0, i, 0))
    out = pl.pallas_call(
        functools.partial(_final_body, n),
        grid=(grid,),
        in_specs=[agg_spec, u_spec, col_spec, col_spec, vec_spec, mat_spec,
                  vec_spec, mat_spec, vec_spec],
        out_specs=pl.BlockSpec((1, d), lambda i: (0, 0)),
        out_shape=jax.ShapeDtypeStruct((1, d), f32),
        scratch_shapes=[pltpu.VMEM((8, d), f32)],
    )(agg2.reshape(NC, n_pad, d), u.reshape(NC, n_pad, 1), a_col, b_col,
      b2.reshape(1, d), W3, b3.reshape(1, d), W_out, b_out.reshape(1, d))
    return out.reshape(d)


# speedup vs baseline: 1.1264x; 1.1201x over previous
import functools

import jax
import jax.numpy as jnp
from jax import lax
from jax.experimental import pallas as pl
from jax.experimental.pallas import tpu as pltpu
from jax.experimental.pallas import tpu_sc as plsc

NC = 2
NS = 16
CHUNK = 128


def _round_up(v, m):
    return (v + m - 1) // m * m


def _make_deg_kernel(n_pad, tot, zp, zpb):
    rt = tot // NS
    mesh = plsc.VectorSubcoreMesh(core_axis_name="c", subcore_axis_name="s")

    @functools.partial(
        pl.kernel,
        out_type=jax.ShapeDtypeStruct((NC * n_pad,), jnp.float32),
        mesh=mesh,
        scratch_types=[
            pltpu.VMEM((rt, CHUNK), jnp.int32),
            pltpu.VMEM((CHUNK,), jnp.float32),
            pltpu.VMEM((zpb,), jnp.float32),
            pltpu.VMEM_SHARED((n_pad,), jnp.float32),
        ],
    )
    def deg_kernel(ecat_hbm, z1_hbm, out_hbm, idx_v, ones_v, zv, acc_s):
        c = lax.axis_index("c")
        s = lax.axis_index("s")
        for j in range(CHUNK // 16):
            ones_v[pl.ds(j * 16, 16)] = jnp.ones((16,), jnp.float32)
        pltpu.sync_copy(z1_hbm, zv)
        base = s * zp
        pltpu.sync_copy(zv.at[pl.ds(0, zp)], acc_s.at[pl.ds(base, zp)])
        pltpu.sync_copy(ecat_hbm.at[pl.ds(c * tot + s * rt, rt)], idx_v)
        plsc.subcore_barrier()

        def body(i, carry):
            pltpu.sync_copy(ones_v, acc_s.at[idx_v.at[i]], add=True)
            return carry

        lax.fori_loop(0, rt, body, 0)
        plsc.subcore_barrier()
        pltpu.sync_copy(acc_s.at[pl.ds(base, zp)], zv.at[pl.ds(0, zp)])
        pltpu.sync_copy(zv.at[pl.ds(0, zp)],
                        out_hbm.at[pl.ds(c * n_pad + base, zp)])

    return deg_kernel


def _make_edge_kernel(n_pad, tot, rw, zp, zpb, d, ec, with_u):
    mesh = plsc.VectorSubcoreMesh(core_axis_name="c", subcore_axis_name="s")
    scratch = [
        pltpu.VMEM((rw, 2 * ec), jnp.int32),
        pltpu.VMEM((rw, 2 * ec), jnp.int32),
        pltpu.VMEM((ec, d), jnp.float32),
        pltpu.VMEM((ec, d), jnp.float32),
        pltpu.VMEM((8, d), jnp.float32),
        pltpu.SemaphoreType.DMA,
        pltpu.SemaphoreType.DMA,
        pltpu.SemaphoreType.DMA,
        pltpu.SemaphoreType.DMA,
        pltpu.VMEM_SHARED((n_pad, d), jnp.float32),
    ]
    out_type = [jax.ShapeDtypeStruct((NC * n_pad, d), jnp.float32)]
    if with_u:
        scratch += [
            pltpu.VMEM((ec,), jnp.float32),
            pltpu.VMEM((ec,), jnp.float32),
            pltpu.VMEM((zpb,), jnp.float32),
            pltpu.SemaphoreType.DMA,
            pltpu.SemaphoreType.DMA,
            pltpu.SemaphoreType.DMA,
            pltpu.SemaphoreType.DMA,
            pltpu.VMEM_SHARED((n_pad,), jnp.float32),
        ]
        out_type.append(jax.ShapeDtypeStruct((NC * n_pad,), jnp.float32))

    mk = functools.partial(pl.kernel, out_type=tuple(out_type), mesh=mesh,
                           scratch_types=scratch)

    def common(c, s, g_hbm, src_hbm, dst_hbm, z2_hbm, agg_hbm, src_v, dst_v,
               bufs, zrow, gsems, ssems, acc_s, uprefetch, uconsume, urefill,
               ufin):
        w = s * NC + c
        base = s * zp
        pltpu.sync_copy(z2_hbm, zrow)

        def zbody(k, carry):
            pltpu.sync_copy(zrow, acc_s.at[pl.ds(base + k * 8, 8)])
            return carry

        lax.fori_loop(0, zp // 8, zbody, 0)
        pltpu.sync_copy(src_hbm.at[pl.ds(w * rw, rw)], src_v)
        pltpu.sync_copy(dst_hbm.at[pl.ds(w * rw, rw)], dst_v)
        plsc.subcore_barrier()

        def sl(ref, i, p):
            return ref.at[i, pl.ds(p * ec, ec)]

        def gath(i, p):
            pltpu.async_copy(g_hbm.at[sl(src_v, i, p)], bufs[p], gsems[p])

        def wait_g(p):
            pltpu.make_async_copy(g_hbm.at[sl(src_v, 0, p)], bufs[p],
                                  gsems[p]).wait()

        def scat(i, p):
            pltpu.async_copy(bufs[p], acc_s.at[sl(dst_v, i, p)], ssems[p],
                             add=True)

        def wait_s(p):
            pltpu.make_async_copy(bufs[p], acc_s.at[sl(dst_v, 0, p)],
                                  ssems[p]).wait()

        gath(0, 0)
        gath(0, 1)
        uprefetch(0, 0)
        uprefetch(0, 1)

        def body(k, carry):
            nxt = jnp.minimum(k + 1, rw - 1)
            wait_g(0)
            scat(k, 0)
            uconsume(k, 0)
            wait_s(0)
            gath(nxt, 0)
            urefill(nxt, 0)
            wait_g(1)
            scat(k, 1)
            uconsume(k, 1)
            wait_s(1)
            gath(nxt, 1)
            urefill(nxt, 1)
            return carry

        lax.fori_loop(0, rw, body, 0)
        for p in (0, 1):
            wait_g(p)
        plsc.subcore_barrier()
        nfull = zp // ec
        rem = zp - nfull * ec
        for k in range(nfull + 1):
            span = ec if k < nfull else rem
            if span:
                pltpu.sync_copy(acc_s.at[pl.ds(base + k * ec, span)],
                                bufs[0].at[pl.ds(0, span)])
                pltpu.sync_copy(bufs[0].at[pl.ds(0, span)],
                                agg_hbm.at[pl.ds(c * n_pad + base + k * ec,
                                                 span)])
        ufin(c, base)

    if with_u:
        @mk
        def edge_kernel(g_hbm, src_hbm, dst_hbm, z2_hbm, b_hbm, z1_hbm,
                        agg_hbm, u_hbm, src_v, dst_v, bufa, bufb, zrow,
                        ga, gb, sa, sb, acc_s, ubufa, ubufb, zv,
                        uga, ugb, usa, usb, uacc_s):
            c = lax.axis_index("c")
            s = lax.axis_index("s")
            base = s * zp
            ubufs, ugs, uss = (ubufa, ubufb), (uga, ugb), (usa, usb)
            pltpu.sync_copy(z1_hbm, zv)
            pltpu.sync_copy(zv.at[pl.ds(0, zp)], uacc_s.at[pl.ds(base, zp)])

            def usl(ref, i, p):
                return ref.at[i, pl.ds(p * ec, ec)]

            def uprefetch(i, p):
                pltpu.async_copy(b_hbm.at[usl(dst_v, i, p)], ubufs[p], ugs[p])

            def uconsume(i, p):
                pltpu.make_async_copy(b_hbm.at[usl(dst_v, i, p)], ubufs[p],
                                      ugs[p]).wait()
                pltpu.async_copy(ubufs[p], uacc_s.at[usl(src_v, i, p)],
                                 uss[p], add=True)

            def urefill(i, p):
                pltpu.make_async_copy(ubufs[p], uacc_s.at[usl(src_v, i, p)],
                                      uss[p]).wait()
                uprefetch(i, p)

            def ufin(cc, bb):
                pltpu.sync_copy(uacc_s.at[pl.ds(bb, zp)], zv.at[pl.ds(0, zp)])
                pltpu.sync_copy(zv.at[pl.ds(0, zp)],
                                u_hbm.at[pl.ds(cc * n_pad + bb, zp)])

            common(c, s, g_hbm, src_hbm, dst_hbm, z2_hbm, agg_hbm,
                   src_v, dst_v, (bufa, bufb), zrow, (ga, gb), (sa, sb),
                   acc_s, uprefetch, uconsume, urefill, ufin)
            for p in (0, 1):
                pltpu.make_async_copy(b_hbm.at[usl(dst_v, 0, p)], ubufs[p],
                                      ugs[p]).wait()
    else:
        @mk
        def edge_kernel(g_hbm, src_hbm, dst_hbm, z2_hbm, agg_hbm,
                        src_v, dst_v, bufa, bufb, zrow, ga, gb, sa, sb,
                        acc_s):
            c = lax.axis_index("c")
            s = lax.axis_index("s")
            common(c, s, g_hbm, src_hbm, dst_hbm, z2_hbm, agg_hbm,
                   src_v, dst_v, (bufa, bufb), zrow, (ga, gb), (sa, sb),
                   acc_s, lambda i, p: None, lambda i, p: None,
                   lambda i, p: None, lambda cc, bb: None)

    return edge_kernel


def _prep_body(n_real, dego_ref, degi_ref, x_ref, win_ref, bin_ref, w1_ref,
               g_ref, a_ref, b_ref):
    i = pl.program_id(0)
    blk = dego_ref.shape[0]
    a = lax.rsqrt(jnp.maximum(dego_ref[...], 1.0))
    rows = lax.broadcasted_iota(jnp.int32, (blk, 1), 0) + i * blk
    b = jnp.where(rows < n_real,
                  lax.rsqrt(jnp.maximum(degi_ref[...], 1.0)), 0.0)
    t = jnp.dot(x_ref[...], win_ref[...],
                preferred_element_type=jnp.float32) + bin_ref[...]
    g = jnp.dot(t, w1_ref[...], preferred_element_type=jnp.float32)
    g_ref[...] = a * g
    a_ref[...] = a
    b_ref[...] = b


def _mid_body(agg_ref, a_ref, b_ref, b1_ref, w2_ref, g_ref):
    ssum = agg_ref[0] + agg_ref[1]
    h = jnp.maximum(b_ref[...] * ssum + b1_ref[...], 0.0)
    g_ref[...] = a_ref[...] * jnp.dot(h, w2_ref[...],
                                      preferred_element_type=jnp.float32)


def _final_body(n_real, agg_ref, u_ref, a_ref, b_ref, b2_ref, w3_ref, b3_ref,
                wo_ref, bo_ref, out_ref, acc_ref):
    i = pl.program_id(0)

    @pl.when(i == 0)
    def _():
        acc_ref[...] = jnp.zeros_like(acc_ref)

    ssum = agg_ref[0] + agg_ref[1]
    h2 = jnp.maximum(b_ref[...] * ssum + b2_ref[...], 0.0)
    wv = a_ref[...] * (u_ref[0] + u_ref[1])
    acc_ref[0:1, :] += jnp.sum(wv * h2, axis=0, keepdims=True)

    @pl.when(i == pl.num_programs(0) - 1)
    def _():
        p = acc_ref[0:1, :]
        t = (jnp.dot(p, w3_ref[...], preferred_element_type=jnp.float32)
             + jnp.float32(n_real) * b3_ref[...])
        out_ref[...] = (jnp.dot(t, wo_ref[...],
                                preferred_element_type=jnp.float32)
                        + bo_ref[...])


def kernel(x, edge_index, W_in, b_in, W1, b1, W2, b2, W3, b3, W_out, b_out):
    n, d = x.shape
    e = edge_index.shape[1]
    nw = NC * NS
    e_pad = _round_up(e, nw * CHUNK * 8)
    tot = e_pad // CHUNK
    ec = 64
    rw = tot // nw
    n_pad = _round_up(n + 16, 128)
    zp = n_pad // NS
    zpb = _round_up(zp, 16)
    blk = 128
    grid = n_pad // blk

    f32 = jnp.float32
    src = edge_index[0]
    dst = edge_index[1]
    padc = e_pad - e
    if padc:
        pidx = (n + (jnp.arange(padc, dtype=jnp.int32) % 16)).astype(jnp.int32)
        src = jnp.concatenate([src, pidx])
        dst = jnp.concatenate([dst, pidx])
    src2 = src.reshape(tot, CHUNK)
    dst2 = dst.reshape(tot, CHUNK)
    ecat = jnp.concatenate([src2, dst2], axis=0)
    z2 = jnp.zeros((8, d), f32)
    z1 = jnp.zeros((zpb,), f32)
    x_pad = jnp.concatenate([x, jnp.zeros((n_pad - n, d), f32)], axis=0)

    deg = _make_deg_kernel(n_pad, tot, zp, zpb)(ecat, z1)
    dego = deg[:n_pad].reshape(n_pad, 1)
    degi = deg[n_pad:].reshape(n_pad, 1)

    col_spec = pl.BlockSpec((blk, 1), lambda i: (i, 0))
    row_spec = pl.BlockSpec((blk, d), lambda i: (i, 0))
    mat_spec = pl.BlockSpec((d, d), lambda i: (0, 0))
    vec_spec = pl.BlockSpec((1, d), lambda i: (0, 0))
    g0p, a_col, b_col = pl.pallas_call(
        functools.partial(_prep_body, n),
        grid=(grid,),
        in_specs=[col_spec, col_spec, row_spec, mat_spec, vec_spec, mat_spec],
        out_specs=[row_spec, col_spec, col_spec],
        out_shape=[jax.ShapeDtypeStruct((n_pad, d), f32),
                   jax.ShapeDtypeStruct((n_pad, 1), f32),
                   jax.ShapeDtypeStruct((n_pad, 1), f32)],
    )(dego, degi, x_pad, W_in, b_in.reshape(1, d), W1)

    agg1, u = _make_edge_kernel(n_pad, tot, rw, zp, zpb, d, ec, True)(
        g0p, src2, dst2, z2, b_col.reshape(n_pad), z1)

    agg_spec = pl.BlockSpec((2, blk, d), lambda i: (0, i, 0))
    g1p = pl.pallas_call(
        _mid_body,
        grid=(grid,),
        in_specs=[agg_spec, col_spec, col_spec, vec_spec, mat_spec],
        out_specs=row_spec,
        out_shape=jax.ShapeDtypeStruct((n_pad, d), f32),
    )(agg1.reshape(NC, n_pad, d), a_col, b_col, b1.reshape(1, d), W2)

    (agg2,) = _make_edge_kernel(n_pad, tot, rw, zp, zpb, d, ec, False)(
        g1p, src2, dst2, z2)

    u_spec = pl.BlockSpec((2, blk, 1), lambda i: (0, i, 0))
    out = pl.pallas_call(
        functools.partial(_final_body, n),
        grid=(grid,),
        in_specs=[agg_spec, u_spec, col_spec, col_spec, vec_spec, mat_spec,
                  vec_spec, mat_spec, vec_spec],
        out_specs=pl.BlockSpec((1, d), lambda i: (0, 0)),
        out_shape=jax.ShapeDtypeStruct((1, d), f32),
        scratch_shapes=[pltpu.VMEM((8, d), f32)],
    )(agg2.reshape(NC, n_pad, d), u.reshape(NC, n_pad, 1), a_col, b_col,
      b2.reshape(1, d), W3, b3.reshape(1, d), W_out, b_out.reshape(1, d))
    return out.reshape(d)


# speedup vs baseline: 1.4677x; 1.3030x over previous
import functools

import jax
import jax.numpy as jnp
from jax import lax
from jax.experimental import pallas as pl
from jax.experimental.pallas import tpu as pltpu
from jax.experimental.pallas import tpu_sc as plsc

NC = 2
NS = 16
CHUNK = 128


def _round_up(v, m):
    return (v + m - 1) // m * m


def _make_deg_kernel(n_pad, tot, zp, zpb):
    rt = tot // NS
    mesh = plsc.VectorSubcoreMesh(core_axis_name="c", subcore_axis_name="s")

    @functools.partial(
        pl.kernel,
        out_type=jax.ShapeDtypeStruct((NC, n_pad), jnp.float32),
        mesh=mesh,
        scratch_types=[
            pltpu.VMEM((rt, CHUNK), jnp.int32),
            pltpu.VMEM((CHUNK,), jnp.float32),
            pltpu.VMEM((zpb,), jnp.float32),
            pltpu.VMEM_SHARED((n_pad,), jnp.float32),
        ],
    )
    def deg_kernel(ecat_hbm, z1_hbm, out_hbm, idx_v, ones_v, zv, acc_s):
        c = lax.axis_index("c")
        s = lax.axis_index("s")
        for j in range(CHUNK // 16):
            ones_v[pl.ds(j * 16, 16)] = jnp.ones((16,), jnp.float32)
        pltpu.sync_copy(z1_hbm, zv)
        base = s * zp
        pltpu.sync_copy(zv.at[pl.ds(0, zp)], acc_s.at[pl.ds(base, zp)])
        pltpu.sync_copy(ecat_hbm.at[pl.ds(c * tot + s * rt, rt)], idx_v)
        plsc.subcore_barrier()

        def body(i, carry):
            pltpu.sync_copy(ones_v, acc_s.at[idx_v.at[i]], add=True)
            return carry

        lax.fori_loop(0, rt, body, 0)
        plsc.subcore_barrier()
        pltpu.sync_copy(acc_s.at[pl.ds(base, zp)], zv.at[pl.ds(0, zp)])
        pltpu.sync_copy(zv.at[pl.ds(0, zp)],
                        out_hbm.at[c].at[pl.ds(base, zp)])

    return deg_kernel


def _make_edge_kernel(n_pad, tot, rw, zp, zpb, d, ec, with_u):
    mesh = plsc.VectorSubcoreMesh(core_axis_name="c", subcore_axis_name="s")
    scratch = [
        pltpu.VMEM((rw, 2 * ec), jnp.int32),
        pltpu.VMEM((rw, 2 * ec), jnp.int32),
        pltpu.VMEM((ec, d), jnp.float32),
        pltpu.VMEM((ec, d), jnp.float32),
        pltpu.VMEM((8, d), jnp.float32),
        pltpu.SemaphoreType.DMA,
        pltpu.SemaphoreType.DMA,
        pltpu.SemaphoreType.DMA,
        pltpu.SemaphoreType.DMA,
        pltpu.VMEM_SHARED((n_pad, d), jnp.float32),
    ]
    out_type = [jax.ShapeDtypeStruct((NC, n_pad, d), jnp.float32)]
    if with_u:
        scratch += [
            pltpu.VMEM((ec,), jnp.float32),
            pltpu.VMEM((ec,), jnp.float32),
            pltpu.VMEM((zpb,), jnp.float32),
            pltpu.SemaphoreType.DMA,
            pltpu.SemaphoreType.DMA,
            pltpu.SemaphoreType.DMA,
            pltpu.SemaphoreType.DMA,
            pltpu.VMEM_SHARED((n_pad,), jnp.float32),
        ]
        out_type.append(jax.ShapeDtypeStruct((NC, n_pad), jnp.float32))

    mk = functools.partial(pl.kernel, out_type=tuple(out_type), mesh=mesh,
                           scratch_types=scratch)

    def common(c, s, g_hbm, src_hbm, dst_hbm, z2_hbm, agg_hbm, src_v, dst_v,
               bufs, zrow, gsems, ssems, acc_s, uprefetch, uconsume, urefill,
               ufin):
        w = s * NC + c
        base = s * zp
        pltpu.sync_copy(z2_hbm, zrow)

        def zbody(k, carry):
            pltpu.sync_copy(zrow, acc_s.at[pl.ds(base + k * 8, 8)])
            return carry

        lax.fori_loop(0, zp // 8, zbody, 0)
        pltpu.sync_copy(src_hbm.at[pl.ds(w * rw, rw)], src_v)
        pltpu.sync_copy(dst_hbm.at[pl.ds(w * rw, rw)], dst_v)
        plsc.subcore_barrier()

        def sl(ref, i, p):
            return ref.at[i, pl.ds(p * ec, ec)]

        def gath(i, p):
            pltpu.async_copy(g_hbm.at[sl(src_v, i, p)], bufs[p], gsems[p])

        def wait_g(p):
            pltpu.make_async_copy(g_hbm.at[sl(src_v, 0, p)], bufs[p],
                                  gsems[p]).wait()

        def scat(i, p):
            pltpu.async_copy(bufs[p], acc_s.at[sl(dst_v, i, p)], ssems[p],
                             add=True)

        def wait_s(p):
            pltpu.make_async_copy(bufs[p], acc_s.at[sl(dst_v, 0, p)],
                                  ssems[p]).wait()

        gath(0, 0)
        gath(0, 1)
        uprefetch(0, 0)
        uprefetch(0, 1)

        def body(k, carry):
            nxt = jnp.minimum(k + 1, rw - 1)
            wait_g(0)
            scat(k, 0)
            uconsume(k, 0)
            wait_s(0)
            gath(nxt, 0)
            urefill(nxt, 0)
            wait_g(1)
            scat(k, 1)
            uconsume(k, 1)
            wait_s(1)
            gath(nxt, 1)
            urefill(nxt, 1)
            return carry

        lax.fori_loop(0, rw, body, 0)
        for p in (0, 1):
            wait_g(p)
        plsc.subcore_barrier()
        nfull = zp // ec
        rem = zp - nfull * ec
        for k in range(nfull + 1):
            span = ec if k < nfull else rem
            if span:
                pltpu.sync_copy(acc_s.at[pl.ds(base + k * ec, span)],
                                bufs[0].at[pl.ds(0, span)])
                pltpu.sync_copy(bufs[0].at[pl.ds(0, span)],
                                agg_hbm.at[c].at[pl.ds(base + k * ec, span)])
        ufin(c, base)

    if with_u:
        @mk
        def edge_kernel(g_hbm, src_hbm, dst_hbm, z2_hbm, b_hbm, z1_hbm,
                        agg_hbm, u_hbm, src_v, dst_v, bufa, bufb, zrow,
                        ga, gb, sa, sb, acc_s, ubufa, ubufb, zv,
                        uga, ugb, usa, usb, uacc_s):
            c = lax.axis_index("c")
            s = lax.axis_index("s")
            base = s * zp
            ubufs, ugs, uss = (ubufa, ubufb), (uga, ugb), (usa, usb)
            pltpu.sync_copy(z1_hbm, zv)
            pltpu.sync_copy(zv.at[pl.ds(0, zp)], uacc_s.at[pl.ds(base, zp)])

            def usl(ref, i, p):
                return ref.at[i, pl.ds(p * ec, ec)]

            def uprefetch(i, p):
                pltpu.async_copy(b_hbm.at[usl(dst_v, i, p)], ubufs[p], ugs[p])

            def uconsume(i, p):
                pltpu.make_async_copy(b_hbm.at[usl(dst_v, i, p)], ubufs[p],
                                      ugs[p]).wait()
                pltpu.async_copy(ubufs[p], uacc_s.at[usl(src_v, i, p)],
                                 uss[p], add=True)

            def urefill(i, p):
                pltpu.make_async_copy(ubufs[p], uacc_s.at[usl(src_v, i, p)],
                                      uss[p]).wait()
                uprefetch(i, p)

            def ufin(cc, bb):
                pltpu.sync_copy(uacc_s.at[pl.ds(bb, zp)], zv.at[pl.ds(0, zp)])
                pltpu.sync_copy(zv.at[pl.ds(0, zp)],
                                u_hbm.at[cc].at[pl.ds(bb, zp)])

            common(c, s, g_hbm, src_hbm, dst_hbm, z2_hbm, agg_hbm,
                   src_v, dst_v, (bufa, bufb), zrow, (ga, gb), (sa, sb),
                   acc_s, uprefetch, uconsume, urefill, ufin)
            for p in (0, 1):
                pltpu.make_async_copy(b_hbm.at[usl(dst_v, 0, p)], ubufs[p],
                                      ugs[p]).wait()
    else:
        @mk
        def edge_kernel(g_hbm, src_hbm, dst_hbm, z2_hbm, agg_hbm,
                        src_v, dst_v, bufa, bufb, zrow, ga, gb, sa, sb,
                        acc_s):
            c = lax.axis_index("c")
            s = lax.axis_index("s")
            common(c, s, g_hbm, src_hbm, dst_hbm, z2_hbm, agg_hbm,
                   src_v, dst_v, (bufa, bufb), zrow, (ga, gb), (sa, sb),
                   acc_s, lambda i, p: None, lambda i, p: None,
                   lambda i, p: None, lambda cc, bb: None)

    return edge_kernel


def _prep_body(n_real, dego_ref, degi_ref, x_ref, win_ref, bin_ref, w1_ref,
               g_ref, a_ref, b_ref):
    i = pl.program_id(0)
    blk = dego_ref.shape[0]
    a = lax.rsqrt(jnp.maximum(dego_ref[...], 1.0))
    rows = lax.broadcasted_iota(jnp.int32, (blk, 1), 0) + i * blk
    b = jnp.where(rows < n_real,
                  lax.rsqrt(jnp.maximum(degi_ref[...], 1.0)), 0.0)
    t = jnp.dot(x_ref[...], win_ref[...],
                preferred_element_type=jnp.float32) + bin_ref[...]
    g = jnp.dot(t, w1_ref[...], preferred_element_type=jnp.float32)
    g_ref[...] = a * g
    a_ref[...] = a
    b_ref[...] = b


def _mid_body(agg_ref, a_ref, b_ref, b1_ref, w2_ref, g_ref):
    ssum = agg_ref[0] + agg_ref[1]
    h = jnp.maximum(b_ref[...] * ssum + b1_ref[...], 0.0)
    g_ref[...] = a_ref[...] * jnp.dot(h, w2_ref[...],
                                      preferred_element_type=jnp.float32)


def _final_body(n_real, agg_ref, u_ref, a_ref, b_ref, b2_ref, w3_ref, b3_ref,
                wo_ref, bo_ref, out_ref, acc_ref):
    i = pl.program_id(0)

    @pl.when(i == 0)
    def _():
        acc_ref[...] = jnp.zeros_like(acc_ref)

    ssum = agg_ref[0] + agg_ref[1]
    h2 = jnp.maximum(b_ref[...] * ssum + b2_ref[...], 0.0)
    wv = a_ref[...] * (u_ref[0] + u_ref[1])
    acc_ref[0:1, :] += jnp.sum(wv * h2, axis=0, keepdims=True)

    @pl.when(i == pl.num_programs(0) - 1)
    def _():
        p = acc_ref[0:1, :]
        t = (jnp.dot(p, w3_ref[...], preferred_element_type=jnp.float32)
             + jnp.float32(n_real) * b3_ref[...])
        out_ref[...] = (jnp.dot(t, wo_ref[...],
                                preferred_element_type=jnp.float32)
                        + bo_ref[...])


def kernel(x, edge_index, W_in, b_in, W1, b1, W2, b2, W3, b3, W_out, b_out):
    n, d = x.shape
    e = edge_index.shape[1]
    nw = NC * NS
    e_pad = _round_up(e, nw * CHUNK * 8)
    tot = e_pad // CHUNK
    ec = 64
    rw = tot // nw
    n_pad = _round_up(n + 16, 1280)
    zp = n_pad // NS
    zpb = _round_up(zp, 16)
    grid = 8
    blk = n_pad // grid

    f32 = jnp.float32
    src = edge_index[0]
    dst = edge_index[1]
    padc = e_pad - e
    if padc:
        pidx = (n + (jnp.arange(padc, dtype=jnp.int32) % 16)).astype(jnp.int32)
        src = jnp.concatenate([src, pidx])
        dst = jnp.concatenate([dst, pidx])
    src2 = src.reshape(tot, CHUNK)
    dst2 = dst.reshape(tot, CHUNK)
    ecat = jnp.concatenate([src2, dst2], axis=0)
    z2 = jnp.zeros((8, d), f32)
    z1 = jnp.zeros((zpb,), f32)
    x_pad = jnp.concatenate([x, jnp.zeros((n_pad - n, d), f32)], axis=0)

    deg = _make_deg_kernel(n_pad, tot, zp, zpb)(ecat, z1)
    dego = deg[0].reshape(n_pad, 1)
    degi = deg[1].reshape(n_pad, 1)

    col_spec = pl.BlockSpec((blk, 1), lambda i: (i, 0))
    row_spec = pl.BlockSpec((blk, d), lambda i: (i, 0))
    mat_spec = pl.BlockSpec((d, d), lambda i: (0, 0))
    vec_spec = pl.BlockSpec((1, d), lambda i: (0, 0))
    g0p, a_col, b_col = pl.pallas_call(
        functools.partial(_prep_body, n),
        grid=(grid,),
        in_specs=[col_spec, col_spec, row_spec, mat_spec, vec_spec, mat_spec],
        out_specs=[row_spec, col_spec, col_spec],
        out_shape=[jax.ShapeDtypeStruct((n_pad, d), f32),
                   jax.ShapeDtypeStruct((n_pad, 1), f32),
                   jax.ShapeDtypeStruct((n_pad, 1), f32)],
    )(dego, degi, x_pad, W_in, b_in.reshape(1, d), W1)

    agg1, u = _make_edge_kernel(n_pad, tot, rw, zp, zpb, d, ec, True)(
        g0p, src2, dst2, z2, b_col.reshape(n_pad), z1)

    agg_spec = pl.BlockSpec((2, blk, d), lambda i: (0, i, 0))
    g1p = pl.pallas_call(
        _mid_body,
        grid=(grid,),
        in_specs=[agg_spec, col_spec, col_spec, vec_spec, mat_spec],
        out_specs=row_spec,
        out_shape=jax.ShapeDtypeStruct((n_pad, d), f32),
    )(agg1, a_col, b_col, b1.reshape(1, d), W2)

    (agg2,) = _make_edge_kernel(n_pad, tot, rw, zp, zpb, d, ec, False)(
        g1p, src2, dst2, z2)

    u_spec = pl.BlockSpec((2, blk, 1), lambda i: (0, i, 0))
    out = pl.pallas_call(
        functools.partial(_final_body, n),
        grid=(grid,),
        in_specs=[agg_spec, u_spec, col_spec, col_spec, vec_spec, mat_spec,
                  vec_spec, mat_spec, vec_spec],
        out_specs=pl.BlockSpec((1, d), lambda i: (0, 0)),
        out_shape=jax.ShapeDtypeStruct((1, d), f32),
        scratch_shapes=[pltpu.VMEM((8, d), f32)],
    )(agg2, u.reshape(NC, n_pad, 1), a_col, b_col,
      b2.reshape(1, d), W3, b3.reshape(1, d), W_out, b_out.reshape(1, d))
    return out.reshape(d)


# speedup vs baseline: 1.5907x; 1.0838x over previous
import functools

import jax
import jax.numpy as jnp
from jax import lax
from jax.experimental import pallas as pl
from jax.experimental.pallas import tpu as pltpu
from jax.experimental.pallas import tpu_sc as plsc

NC = 2
NS = 16
CHUNK = 128


def _round_up(v, m):
    return (v + m - 1) // m * m


def _make_deg_kernel(n_pad, tot, zp, zpb):
    rt = tot // NS
    mesh = plsc.VectorSubcoreMesh(core_axis_name="c", subcore_axis_name="s")

    @functools.partial(
        pl.kernel,
        out_type=jax.ShapeDtypeStruct((NC, n_pad), jnp.float32),
        mesh=mesh,
        scratch_types=[
            pltpu.VMEM((rt, CHUNK), jnp.int32),
            pltpu.VMEM((CHUNK,), jnp.float32),
            pltpu.VMEM((zpb,), jnp.float32),
            pltpu.VMEM_SHARED((n_pad,), jnp.float32),
        ],
    )
    def deg_kernel(ecat_hbm, z1_hbm, out_hbm, idx_v, ones_v, zv, acc_s):
        c = lax.axis_index("c")
        s = lax.axis_index("s")
        for j in range(CHUNK // 16):
            ones_v[pl.ds(j * 16, 16)] = jnp.ones((16,), jnp.float32)
        pltpu.sync_copy(z1_hbm, zv)
        base = s * zp
        pltpu.sync_copy(zv.at[pl.ds(0, zp)], acc_s.at[pl.ds(base, zp)])
        pltpu.sync_copy(ecat_hbm.at[pl.ds(c * tot + s * rt, rt)], idx_v)
        plsc.subcore_barrier()

        def body(i, carry):
            pltpu.sync_copy(ones_v, acc_s.at[idx_v.at[i]], add=True)
            return carry

        lax.fori_loop(0, rt, body, 0)
        plsc.subcore_barrier()
        pltpu.sync_copy(acc_s.at[pl.ds(base, zp)], zv.at[pl.ds(0, zp)])
        pltpu.sync_copy(zv.at[pl.ds(0, zp)],
                        out_hbm.at[c].at[pl.ds(base, zp)])

    return deg_kernel


def _make_edge_kernel(n_pad, tot, rw, zp, zpb, d, ec, with_u):
    mesh = plsc.VectorSubcoreMesh(core_axis_name="c", subcore_axis_name="s")
    scratch = [
        pltpu.VMEM((8, ec), jnp.int32),
        pltpu.VMEM((8, ec), jnp.int32),
        pltpu.VMEM((8, ec), jnp.int32),
        pltpu.VMEM((8, ec), jnp.int32),
        pltpu.VMEM((ec, d), jnp.float32),
        pltpu.VMEM((ec, d), jnp.float32),
        pltpu.VMEM((8, d), jnp.float32),
        pltpu.SemaphoreType.DMA,
        pltpu.SemaphoreType.DMA,
        pltpu.SemaphoreType.DMA,
        pltpu.SemaphoreType.DMA,
        pltpu.SemaphoreType.DMA,
        pltpu.SemaphoreType.DMA,
        pltpu.VMEM_SHARED((n_pad, d), jnp.float32),
    ]
    out_type = [jax.ShapeDtypeStruct((NC, n_pad, d), jnp.float32)]
    if with_u:
        scratch += [
            pltpu.VMEM((ec,), jnp.float32),
            pltpu.VMEM((ec,), jnp.float32),
            pltpu.VMEM((zpb,), jnp.float32),
            pltpu.SemaphoreType.DMA,
            pltpu.SemaphoreType.DMA,
            pltpu.SemaphoreType.DMA,
            pltpu.SemaphoreType.DMA,
            pltpu.VMEM_SHARED((n_pad,), jnp.float32),
        ]
        out_type.append(jax.ShapeDtypeStruct((NC, n_pad), jnp.float32))

    mk = functools.partial(pl.kernel, out_type=tuple(out_type), mesh=mesh,
                           scratch_types=scratch)

    def common(c, s, g_hbm, src_hbm, dst_hbm, z2_hbm, agg_hbm, svs, dvs,
               bufs, zrow, isems, gsems, ssems, acc_s, uprefetch, uconsume,
               urefill, ufin):
        w = s * NC + c
        base = s * zp
        wbase = w * rw
        nb = rw // 8
        pltpu.sync_copy(z2_hbm, zrow)

        def zbody(k, carry):
            pltpu.sync_copy(zrow, acc_s.at[pl.ds(base + k * 8, 8)])
            return carry

        lax.fori_loop(0, zp // 8, zbody, 0)

        def load_idx(q, bb):
            off = wbase + bb * 8
            pltpu.async_copy(src_hbm.at[pl.ds(off, 8)], svs[q], isems[q])
            pltpu.async_copy(dst_hbm.at[pl.ds(off, 8)], dvs[q], isems[q])

        def wait_idx(q):
            pltpu.make_async_copy(src_hbm.at[pl.ds(wbase, 8)], svs[q],
                                  isems[q]).wait()
            pltpu.make_async_copy(dst_hbm.at[pl.ds(wbase, 8)], dvs[q],
                                  isems[q]).wait()

        pltpu.sync_copy(src_hbm.at[pl.ds(wbase, 8)], svs[0])
        pltpu.sync_copy(dst_hbm.at[pl.ds(wbase, 8)], dvs[0])
        load_idx(1, 1)
        plsc.subcore_barrier()

        def gath(sv, j, p):
            pltpu.async_copy(g_hbm.at[sv.at[j]], bufs[p], gsems[p])

        def wait_g(p):
            pltpu.make_async_copy(g_hbm.at[svs[0].at[0]], bufs[p],
                                  gsems[p]).wait()

        def scat(dv, j, p):
            pltpu.async_copy(bufs[p], acc_s.at[dv.at[j]], ssems[p], add=True)

        def wait_s(p):
            pltpu.make_async_copy(bufs[p], acc_s.at[dvs[0].at[0]],
                                  ssems[p]).wait()

        gath(svs[0], 0, 0)
        gath(svs[0], 1, 1)
        uprefetch(dvs[0], 0, 0)
        uprefetch(dvs[0], 1, 1)

        def process_block(q):
            sv, dv = svs[q], dvs[q]
            svn, dvn = svs[1 - q], dvs[1 - q]
            for j in range(8):
                p = j & 1
                if j == 6:
                    wait_idx(1 - q)
                wait_g(p)
                scat(dv, j, p)
                uconsume(dv, sv, j, p)
                wait_s(p)
                if j < 6:
                    gath(sv, j + 2, p)
                    urefill(dv, j + 2, p)
                else:
                    gath(svn, j - 6, p)
                    urefill(dvn, j - 6, p)

        def body(b2, carry):
            bb0 = 2 * b2
            process_block(0)
            load_idx(0, jnp.minimum(bb0 + 2, nb - 1))
            process_block(1)
            load_idx(1, jnp.minimum(bb0 + 3, nb - 1))
            return carry

        lax.fori_loop(0, nb // 2, body, 0)
        for p in (0, 1):
            wait_g(p)
        wait_idx(1)
        plsc.subcore_barrier()
        nfull = zp // ec
        rem = zp - nfull * ec
        for k in range(nfull + 1):
            span = ec if k < nfull else rem
            if span:
                pltpu.sync_copy(acc_s.at[pl.ds(base + k * ec, span)],
                                bufs[0].at[pl.ds(0, span)])
                pltpu.sync_copy(bufs[0].at[pl.ds(0, span)],
                                agg_hbm.at[c].at[pl.ds(base + k * ec, span)])
        ufin(c, base)

    if with_u:
        @mk
        def edge_kernel(g_hbm, src_hbm, dst_hbm, z2_hbm, b_hbm, z1_hbm,
                        agg_hbm, u_hbm, sv0, sv1, dv0, dv1, bufa, bufb, zrow,
                        i0, i1, ga, gb, sa, sb, acc_s, ubufa, ubufb, zv,
                        uga, ugb, usa, usb, uacc_s):
            c = lax.axis_index("c")
            s = lax.axis_index("s")
            base = s * zp
            ubufs, ugs, uss = (ubufa, ubufb), (uga, ugb), (usa, usb)
            pltpu.sync_copy(z1_hbm, zv)
            pltpu.sync_copy(zv.at[pl.ds(0, zp)], uacc_s.at[pl.ds(base, zp)])

            def uprefetch(dv, j, p):
                pltpu.async_copy(b_hbm.at[dv.at[j]], ubufs[p], ugs[p])

            def uconsume(dv, sv, j, p):
                pltpu.make_async_copy(b_hbm.at[dv.at[j]], ubufs[p],
                                      ugs[p]).wait()
                pltpu.async_copy(ubufs[p], uacc_s.at[sv.at[j]],
                                 uss[p], add=True)

            def urefill(dv, j, p):
                pltpu.make_async_copy(ubufs[p], uacc_s.at[sv0.at[0]],
                                      uss[p]).wait()
                uprefetch(dv, j, p)

            def ufin(cc, bb):
                pltpu.sync_copy(uacc_s.at[pl.ds(bb, zp)], zv.at[pl.ds(0, zp)])
                pltpu.sync_copy(zv.at[pl.ds(0, zp)],
                                u_hbm.at[cc].at[pl.ds(bb, zp)])

            common(c, s, g_hbm, src_hbm, dst_hbm, z2_hbm, agg_hbm,
                   (sv0, sv1), (dv0, dv1), (bufa, bufb), zrow, (i0, i1),
                   (ga, gb), (sa, sb), acc_s, uprefetch, uconsume, urefill,
                   ufin)
            for p in (0, 1):
                pltpu.make_async_copy(b_hbm.at[dv0.at[0]], ubufs[p],
                                      ugs[p]).wait()
    else:
        @mk
        def edge_kernel(g_hbm, src_hbm, dst_hbm, z2_hbm, agg_hbm,
                        sv0, sv1, dv0, dv1, bufa, bufb, zrow,
                        i0, i1, ga, gb, sa, sb, acc_s):
            c = lax.axis_index("c")
            s = lax.axis_index("s")
            common(c, s, g_hbm, src_hbm, dst_hbm, z2_hbm, agg_hbm,
                   (sv0, sv1), (dv0, dv1), (bufa, bufb), zrow, (i0, i1),
                   (ga, gb), (sa, sb), acc_s,
                   lambda dv, j, p: None, lambda dv, sv, j, p: None,
                   lambda dv, j, p: None, lambda cc, bb: None)

    return edge_kernel


def _prep_body(n_real, dego_ref, degi_ref, x_ref, win_ref, bin_ref, w1_ref,
               g_ref, a_ref, b_ref):
    i = pl.program_id(0)
    blk = dego_ref.shape[0]
    a = lax.rsqrt(jnp.maximum(dego_ref[...], 1.0))
    rows = lax.broadcasted_iota(jnp.int32, (blk, 1), 0) + i * blk
    b = jnp.where(rows < n_real,
                  lax.rsqrt(jnp.maximum(degi_ref[...], 1.0)), 0.0)
    t = jnp.dot(x_ref[...], win_ref[...],
                preferred_element_type=jnp.float32) + bin_ref[...]
    g = jnp.dot(t, w1_ref[...], preferred_element_type=jnp.float32)
    g_ref[...] = a * g
    a_ref[...] = a
    b_ref[...] = b


def _mid_body(agg_ref, a_ref, b_ref, b1_ref, w2_ref, g_ref):
    ssum = agg_ref[0] + agg_ref[1]
    h = jnp.maximum(b_ref[...] * ssum + b1_ref[...], 0.0)
    g_ref[...] = a_ref[...] * jnp.dot(h, w2_ref[...],
                                      preferred_element_type=jnp.float32)


def _final_body(n_real, agg_ref, u_ref, a_ref, b_ref, b2_ref, w3_ref, b3_ref,
                wo_ref, bo_ref, out_ref, acc_ref):
    i = pl.program_id(0)

    @pl.when(i == 0)
    def _():
        acc_ref[...] = jnp.zeros_like(acc_ref)

    ssum = agg_ref[0] + agg_ref[1]
    h2 = jnp.maximum(b_ref[...] * ssum + b2_ref[...], 0.0)
    wv = a_ref[...] * (u_ref[0] + u_ref[1])
    acc_ref[0:1, :] += jnp.sum(wv * h2, axis=0, keepdims=True)

    @pl.when(i == pl.num_programs(0) - 1)
    def _():
        p = acc_ref[0:1, :]
        t = (jnp.dot(p, w3_ref[...], preferred_element_type=jnp.float32)
             + jnp.float32(n_real) * b3_ref[...])
        out_ref[...] = (jnp.dot(t, wo_ref[...],
                                preferred_element_type=jnp.float32)
                        + bo_ref[...])


def kernel(x, edge_index, W_in, b_in, W1, b1, W2, b2, W3, b3, W_out, b_out):
    n, d = x.shape
    e = edge_index.shape[1]
    nw = NC * NS
    e_pad = _round_up(e, nw * CHUNK * 8)
    tot = e_pad // CHUNK
    ec = CHUNK
    rw = tot // nw
    n_pad = _round_up(n + 16, 1280)
    zp = n_pad // NS
    zpb = _round_up(zp, 16)
    grid = 8
    blk = n_pad // grid

    f32 = jnp.float32
    src = edge_index[0]
    dst = edge_index[1]
    padc = e_pad - e
    if padc:
        pidx = (n + (jnp.arange(padc, dtype=jnp.int32) % 16)).astype(jnp.int32)
        src = jnp.concatenate([src, pidx])
        dst = jnp.concatenate([dst, pidx])
    src2 = src.reshape(tot, CHUNK)
    dst2 = dst.reshape(tot, CHUNK)
    ecat = jnp.concatenate([src2, dst2], axis=0)
    z2 = jnp.zeros((8, d), f32)
    z1 = jnp.zeros((zpb,), f32)
    x_pad = jnp.concatenate([x, jnp.zeros((n_pad - n, d), f32)], axis=0)

    deg = _make_deg_kernel(n_pad, tot, zp, zpb)(ecat, z1)
    dego = deg[0].reshape(n_pad, 1)
    degi = deg[1].reshape(n_pad, 1)

    col_spec = pl.BlockSpec((blk, 1), lambda i: (i, 0))
    row_spec = pl.BlockSpec((blk, d), lambda i: (i, 0))
    mat_spec = pl.BlockSpec((d, d), lambda i: (0, 0))
    vec_spec = pl.BlockSpec((1, d), lambda i: (0, 0))
    g0p, a_col, b_col = pl.pallas_call(
        functools.partial(_prep_body, n),
        grid=(grid,),
        in_specs=[col_spec, col_spec, row_spec, mat_spec, vec_spec, mat_spec],
        out_specs=[row_spec, col_spec, col_spec],
        out_shape=[jax.ShapeDtypeStruct((n_pad, d), f32),
                   jax.ShapeDtypeStruct((n_pad, 1), f32),
                   jax.ShapeDtypeStruct((n_pad, 1), f32)],
    )(dego, degi, x_pad, W_in, b_in.reshape(1, d), W1)

    agg1, u = _make_edge_kernel(n_pad, tot, rw, zp, zpb, d, ec, True)(
        g0p, src2, dst2, z2, b_col.reshape(n_pad), z1)

    agg_spec = pl.BlockSpec((2, blk, d), lambda i: (0, i, 0))
    g1p = pl.pallas_call(
        _mid_body,
        grid=(grid,),
        in_specs=[agg_spec, col_spec, col_spec, vec_spec, mat_spec],
        out_specs=row_spec,
        out_shape=jax.ShapeDtypeStruct((n_pad, d), f32),
    )(agg1, a_col, b_col, b1.reshape(1, d), W2)

    (agg2,) = _make_edge_kernel(n_pad, tot, rw, zp, zpb, d, ec, False)(
        g1p, src2, dst2, z2)

    u_spec = pl.BlockSpec((2, blk, 1), lambda i: (0, i, 0))
    out = pl.pallas_call(
        functools.partial(_final_body, n),
        grid=(grid,),
        in_specs=[agg_spec, u_spec, col_spec, col_spec, vec_spec, mat_spec,
                  vec_spec, mat_spec, vec_spec],
        out_specs=pl.BlockSpec((1, d), lambda i: (0, 0)),
        out_shape=jax.ShapeDtypeStruct((1, d), f32),
        scratch_shapes=[pltpu.VMEM((8, d), f32)],
    )(agg2, u.reshape(NC, n_pad, 1), a_col, b_col,
      b2.reshape(1, d), W3, b3.reshape(1, d), W_out, b_out.reshape(1, d))
    return out.reshape(d)


# speedup vs baseline: 1.6142x; 1.0148x over previous
import functools

import jax
import jax.numpy as jnp
from jax import lax
from jax.experimental import pallas as pl
from jax.experimental.pallas import tpu as pltpu
from jax.experimental.pallas import tpu_sc as plsc

NC = 2
NS = 16
CHUNK = 128


def _round_up(v, m):
    return (v + m - 1) // m * m


def _make_deg_kernel(n_pad, tot, zp, zpb):
    rt = tot // NS
    mesh = plsc.VectorSubcoreMesh(core_axis_name="c", subcore_axis_name="s")

    @functools.partial(
        pl.kernel,
        out_type=jax.ShapeDtypeStruct((NC, n_pad), jnp.float32),
        mesh=mesh,
        scratch_types=[
            pltpu.VMEM((rt, CHUNK), jnp.int32),
            pltpu.VMEM((CHUNK,), jnp.float32),
            pltpu.VMEM((zpb,), jnp.float32),
            pltpu.VMEM_SHARED((n_pad,), jnp.float32),
        ],
    )
    def deg_kernel(src_hbm, dst_hbm, z1_hbm, out_hbm, idx_v, ones_v, zv,
                   acc_s):
        c = lax.axis_index("c")
        s = lax.axis_index("s")
        for j in range(CHUNK // 16):
            ones_v[pl.ds(j * 16, 16)] = jnp.ones((16,), jnp.float32)
        pltpu.sync_copy(z1_hbm, zv)
        base = s * zp
        pltpu.sync_copy(zv.at[pl.ds(0, zp)], acc_s.at[pl.ds(base, zp)])

        @pl.when(c == 0)
        def _():
            pltpu.sync_copy(src_hbm.at[pl.ds(s * rt, rt)], idx_v)

        @pl.when(c == 1)
        def _():
            pltpu.sync_copy(dst_hbm.at[pl.ds(s * rt, rt)], idx_v)

        plsc.subcore_barrier()

        def body(i, carry):
            pltpu.sync_copy(ones_v, acc_s.at[idx_v.at[i]], add=True)
            return carry

        lax.fori_loop(0, rt, body, 0)
        plsc.subcore_barrier()
        pltpu.sync_copy(acc_s.at[pl.ds(base, zp)], zv.at[pl.ds(0, zp)])
        pltpu.sync_copy(zv.at[pl.ds(0, zp)],
                        out_hbm.at[c].at[pl.ds(base, zp)])

    return deg_kernel


def _make_edge_kernel(n_pad, tot, rw, zp, zpb, d, ec, with_u):
    mesh = plsc.VectorSubcoreMesh(core_axis_name="c", subcore_axis_name="s")
    scratch = [
        pltpu.VMEM((8, ec), jnp.int32),
        pltpu.VMEM((8, ec), jnp.int32),
        pltpu.VMEM((8, ec), jnp.int32),
        pltpu.VMEM((8, ec), jnp.int32),
        pltpu.VMEM((ec, d), jnp.float32),
        pltpu.VMEM((ec, d), jnp.float32),
        pltpu.VMEM((8, d), jnp.float32),
        pltpu.SemaphoreType.DMA,
        pltpu.SemaphoreType.DMA,
        pltpu.SemaphoreType.DMA,
        pltpu.SemaphoreType.DMA,
        pltpu.SemaphoreType.DMA,
        pltpu.SemaphoreType.DMA,
        pltpu.VMEM_SHARED((n_pad, d), jnp.float32),
    ]
    out_type = [jax.ShapeDtypeStruct((NC, n_pad, d), jnp.float32)]
    if with_u:
        scratch += [
            pltpu.VMEM((ec,), jnp.float32),
            pltpu.VMEM((ec,), jnp.float32),
            pltpu.VMEM((zpb,), jnp.float32),
            pltpu.SemaphoreType.DMA,
            pltpu.SemaphoreType.DMA,
            pltpu.SemaphoreType.DMA,
            pltpu.SemaphoreType.DMA,
            pltpu.VMEM_SHARED((n_pad,), jnp.float32),
        ]
        out_type.append(jax.ShapeDtypeStruct((NC, n_pad), jnp.float32))

    mk = functools.partial(pl.kernel, out_type=tuple(out_type), mesh=mesh,
                           scratch_types=scratch)

    def common(c, s, g_hbm, src_hbm, dst_hbm, z2_hbm, agg_hbm, svs, dvs,
               bufs, zrow, isems, gsems, ssems, acc_s, uprefetch, uconsume,
               urefill, ufin):
        w = s * NC + c
        base = s * zp
        wbase = w * rw
        nb = rw // 8
        pltpu.sync_copy(z2_hbm, zrow)

        def zbody(k, carry):
            pltpu.sync_copy(zrow, acc_s.at[pl.ds(base + k * 8, 8)])
            return carry

        lax.fori_loop(0, zp // 8, zbody, 0)

        def load_idx(q, bb):
            off = wbase + bb * 8
            pltpu.async_copy(src_hbm.at[pl.ds(off, 8)], svs[q], isems[q])
            pltpu.async_copy(dst_hbm.at[pl.ds(off, 8)], dvs[q], isems[q])

        def wait_idx(q):
            pltpu.make_async_copy(src_hbm.at[pl.ds(wbase, 8)], svs[q],
                                  isems[q]).wait()
            pltpu.make_async_copy(dst_hbm.at[pl.ds(wbase, 8)], dvs[q],
                                  isems[q]).wait()

        pltpu.sync_copy(src_hbm.at[pl.ds(wbase, 8)], svs[0])
        pltpu.sync_copy(dst_hbm.at[pl.ds(wbase, 8)], dvs[0])
        load_idx(1, 1)
        plsc.subcore_barrier()

        def gath(sv, j, p):
            pltpu.async_copy(g_hbm.at[sv.at[j]], bufs[p], gsems[p])

        def wait_g(p):
            pltpu.make_async_copy(g_hbm.at[svs[0].at[0]], bufs[p],
                                  gsems[p]).wait()

        def scat(dv, j, p):
            pltpu.async_copy(bufs[p], acc_s.at[dv.at[j]], ssems[p], add=True)

        def wait_s(p):
            pltpu.make_async_copy(bufs[p], acc_s.at[dvs[0].at[0]],
                                  ssems[p]).wait()

        gath(svs[0], 0, 0)
        gath(svs[0], 1, 1)
        uprefetch(dvs[0], 0, 0)
        uprefetch(dvs[0], 1, 1)

        def process_block(q):
            sv, dv = svs[q], dvs[q]
            svn, dvn = svs[1 - q], dvs[1 - q]
            for j in range(8):
                p = j & 1
                if j == 6:
                    wait_idx(1 - q)
                wait_g(p)
                scat(dv, j, p)
                uconsume(dv, sv, j, p)
                wait_s(p)
                if j < 6:
                    gath(sv, j + 2, p)
                    urefill(dv, j + 2, p)
                else:
                    gath(svn, j - 6, p)
                    urefill(dvn, j - 6, p)

        def body(b2, carry):
            bb0 = 2 * b2
            process_block(0)
            load_idx(0, jnp.minimum(bb0 + 2, nb - 1))
            process_block(1)
            load_idx(1, jnp.minimum(bb0 + 3, nb - 1))
            return carry

        lax.fori_loop(0, nb // 2, body, 0)
        for p in (0, 1):
            wait_g(p)
        wait_idx(1)
        plsc.subcore_barrier()
        nfull = zp // ec
        rem = zp - nfull * ec
        for k in range(nfull + 1):
            span = ec if k < nfull else rem
            if span:
                pltpu.sync_copy(acc_s.at[pl.ds(base + k * ec, span)],
                                bufs[0].at[pl.ds(0, span)])
                pltpu.sync_copy(bufs[0].at[pl.ds(0, span)],
                                agg_hbm.at[c].at[pl.ds(base + k * ec, span)])
        ufin(c, base)

    if with_u:
        @mk
        def edge_kernel(g_hbm, src_hbm, dst_hbm, z2_hbm, b_hbm, z1_hbm,
                        agg_hbm, u_hbm, sv0, sv1, dv0, dv1, bufa, bufb, zrow,
                        i0, i1, ga, gb, sa, sb, acc_s, ubufa, ubufb, zv,
                        uga, ugb, usa, usb, uacc_s):
            c = lax.axis_index("c")
            s = lax.axis_index("s")
            base = s * zp
            ubufs, ugs, uss = (ubufa, ubufb), (uga, ugb), (usa, usb)
            pltpu.sync_copy(z1_hbm, zv)
            pltpu.sync_copy(zv.at[pl.ds(0, zp)], uacc_s.at[pl.ds(base, zp)])

            def uprefetch(dv, j, p):
                pltpu.async_copy(b_hbm.at[dv.at[j]], ubufs[p], ugs[p])

            def uconsume(dv, sv, j, p):
                pltpu.make_async_copy(b_hbm.at[dv.at[j]], ubufs[p],
                                      ugs[p]).wait()
                pltpu.async_copy(ubufs[p], uacc_s.at[sv.at[j]],
                                 uss[p], add=True)

            def urefill(dv, j, p):
                pltpu.make_async_copy(ubufs[p], uacc_s.at[sv0.at[0]],
                                      uss[p]).wait()
                uprefetch(dv, j, p)

            def ufin(cc, bb):
                pltpu.sync_copy(uacc_s.at[pl.ds(bb, zp)], zv.at[pl.ds(0, zp)])
                pltpu.sync_copy(zv.at[pl.ds(0, zp)],
                                u_hbm.at[cc].at[pl.ds(bb, zp)])

            common(c, s, g_hbm, src_hbm, dst_hbm, z2_hbm, agg_hbm,
                   (sv0, sv1), (dv0, dv1), (bufa, bufb), zrow, (i0, i1),
                   (ga, gb), (sa, sb), acc_s, uprefetch, uconsume, urefill,
                   ufin)
            for p in (0, 1):
                pltpu.make_async_copy(b_hbm.at[dv0.at[0]], ubufs[p],
                                      ugs[p]).wait()
    else:
        @mk
        def edge_kernel(g_hbm, src_hbm, dst_hbm, z2_hbm, agg_hbm,
                        sv0, sv1, dv0, dv1, bufa, bufb, zrow,
                        i0, i1, ga, gb, sa, sb, acc_s):
            c = lax.axis_index("c")
            s = lax.axis_index("s")
            common(c, s, g_hbm, src_hbm, dst_hbm, z2_hbm, agg_hbm,
                   (sv0, sv1), (dv0, dv1), (bufa, bufb), zrow, (i0, i1),
                   (ga, gb), (sa, sb), acc_s,
                   lambda dv, j, p: None, lambda dv, sv, j, p: None,
                   lambda dv, j, p: None, lambda cc, bb: None)

    return edge_kernel


def _premul_body(x_ref, win_ref, bin_ref, w1_ref, g_ref):
    t = jnp.dot(x_ref[...], win_ref[...],
                preferred_element_type=jnp.float32) + bin_ref[...]
    g_ref[...] = jnp.dot(t, w1_ref[...], preferred_element_type=jnp.float32)


def _scale_body(n_real, dego_ref, degi_ref, gu_ref, g_ref, a_ref, b_ref):
    i = pl.program_id(0)
    blk = dego_ref.shape[0]
    a = lax.rsqrt(jnp.maximum(dego_ref[...], 1.0))
    rows = lax.broadcasted_iota(jnp.int32, (blk, 1), 0) + i * blk
    b = jnp.where(rows < n_real,
                  lax.rsqrt(jnp.maximum(degi_ref[...], 1.0)), 0.0)
    g_ref[...] = a * gu_ref[...]
    a_ref[...] = a
    b_ref[...] = b


def _mid_body(agg_ref, a_ref, b_ref, b1_ref, w2_ref, g_ref):
    ssum = agg_ref[0] + agg_ref[1]
    h = jnp.maximum(b_ref[...] * ssum + b1_ref[...], 0.0)
    g_ref[...] = a_ref[...] * jnp.dot(h, w2_ref[...],
                                      preferred_element_type=jnp.float32)


def _final_body(n_real, agg_ref, u_ref, a_ref, b_ref, b2_ref, w3_ref, b3_ref,
                wo_ref, bo_ref, out_ref, acc_ref):
    i = pl.program_id(0)

    @pl.when(i == 0)
    def _():
        acc_ref[...] = jnp.zeros_like(acc_ref)

    ssum = agg_ref[0] + agg_ref[1]
    h2 = jnp.maximum(b_ref[...] * ssum + b2_ref[...], 0.0)
    wv = a_ref[...] * (u_ref[0] + u_ref[1])
    acc_ref[0:1, :] += jnp.sum(wv * h2, axis=0, keepdims=True)

    @pl.when(i == pl.num_programs(0) - 1)
    def _():
        p = acc_ref[0:1, :]
        t = (jnp.dot(p, w3_ref[...], preferred_element_type=jnp.float32)
             + jnp.float32(n_real) * b3_ref[...])
        out_ref[...] = (jnp.dot(t, wo_ref[...],
                                preferred_element_type=jnp.float32)
                        + bo_ref[...])


def kernel(x, edge_index, W_in, b_in, W1, b1, W2, b2, W3, b3, W_out, b_out):
    n, d = x.shape
    e = edge_index.shape[1]
    nw = NC * NS
    e_pad = _round_up(e, nw * CHUNK * 8)
    tot = e_pad // CHUNK
    ec = CHUNK
    rw = tot // nw
    n_pad = _round_up(n + 16, 1280)
    zp = n_pad // NS
    zpb = _round_up(zp, 16)
    grid = 8
    blk = n_pad // grid

    f32 = jnp.float32
    src = edge_index[0]
    dst = edge_index[1]
    padc = e_pad - e
    if padc:
        pidx = (n + (jnp.arange(padc, dtype=jnp.int32) % 16)).astype(jnp.int32)
        src = jnp.concatenate([src, pidx])
        dst = jnp.concatenate([dst, pidx])
    src2 = src.reshape(tot, CHUNK)
    dst2 = dst.reshape(tot, CHUNK)
    z2 = jnp.zeros((8, d), f32)
    z1 = jnp.zeros((zpb,), f32)
    x_pad = jnp.concatenate([x, jnp.zeros((n_pad - n, d), f32)], axis=0)

    deg = _make_deg_kernel(n_pad, tot, zp, zpb)(src2, dst2, z1)
    dego = deg[0].reshape(n_pad, 1)
    degi = deg[1].reshape(n_pad, 1)

    col_spec = pl.BlockSpec((blk, 1), lambda i: (i, 0))
    row_spec = pl.BlockSpec((blk, d), lambda i: (i, 0))
    mat_spec = pl.BlockSpec((d, d), lambda i: (0, 0))
    vec_spec = pl.BlockSpec((1, d), lambda i: (0, 0))
    g0u = pl.pallas_call(
        _premul_body,
        grid=(grid,),
        in_specs=[row_spec, mat_spec, vec_spec, mat_spec],
        out_specs=row_spec,
        out_shape=jax.ShapeDtypeStruct((n_pad, d), f32),
    )(x_pad, W_in, b_in.reshape(1, d), W1)

    g0p, a_col, b_col = pl.pallas_call(
        functools.partial(_scale_body, n),
        grid=(grid,),
        in_specs=[col_spec, col_spec, row_spec],
        out_specs=[row_spec, col_spec, col_spec],
        out_shape=[jax.ShapeDtypeStruct((n_pad, d), f32),
                   jax.ShapeDtypeStruct((n_pad, 1), f32),
                   jax.ShapeDtypeStruct((n_pad, 1), f32)],
    )(dego, degi, g0u)

    agg1, u = _make_edge_kernel(n_pad, tot, rw, zp, zpb, d, ec, True)(
        g0p, src2, dst2, z2, b_col.reshape(n_pad), z1)

    agg_spec = pl.BlockSpec((2, blk, d), lambda i: (0, i, 0))
    g1p = pl.pallas_call(
        _mid_body,
        grid=(grid,),
        in_specs=[agg_spec, col_spec, col_spec, vec_spec, mat_spec],
        out_specs=row_spec,
        out_shape=jax.ShapeDtypeStruct((n_pad, d), f32),
    )(agg1, a_col, b_col, b1.reshape(1, d), W2)

    (agg2,) = _make_edge_kernel(n_pad, tot, rw, zp, zpb, d, ec, False)(
        g1p, src2, dst2, z2)

    u_spec = pl.BlockSpec((2, blk, 1), lambda i: (0, i, 0))
    out = pl.pallas_call(
        functools.partial(_final_body, n),
        grid=(grid,),
        in_specs=[agg_spec, u_spec, col_spec, col_spec, vec_spec, mat_spec,
                  vec_spec, mat_spec, vec_spec],
        out_specs=pl.BlockSpec((1, d), lambda i: (0, 0)),
        out_shape=jax.ShapeDtypeStruct((1, d), f32),
        scratch_shapes=[pltpu.VMEM((8, d), f32)],
    )(agg2, u.reshape(NC, n_pad, 1), a_col, b_col,
      b2.reshape(1, d), W3, b3.reshape(1, d), W_out, b_out.reshape(1, d))
    return out.reshape(d)
